# baseline (speedup 1.0000x reference)
import functools

import jax
import jax.numpy as jnp
from jax import lax
from jax.experimental import pallas as pl
from jax.experimental.pallas import tpu as pltpu
from jax.experimental.pallas import tpu_sc as plsc

ZERO_WEIGHT = 2.0

_N = 2 * 8192 * 2048
_COLS = 2048
_ROWS = _N // _COLS
_NW = 32
_CROWS = 8
_L = 16
_UNROLL = 8

_SC_CHUNKS_PER_W = 22
_W_ROWS = _SC_CHUNKS_PER_W * _CROWS
_SC_ROWS = _NW * _W_ROWS
_TC_ROWS = _ROWS - _SC_ROWS

_TC_BLOCK_ROWS = 512
_TC_GRID = _TC_ROWS // _TC_BLOCK_ROWS
assert _TC_ROWS % _TC_BLOCK_ROWS == 0

_mesh = plsc.VectorSubcoreMesh(core_axis_name="c", subcore_axis_name="s")


def _tree_sum(vs):
    while len(vs) > 1:
        vs = [a + b for a, b in zip(vs[::2], vs[1::2])]
    return vs[0]


@functools.partial(
    pl.kernel,
    out_type=jax.ShapeDtypeStruct((_NW, 3 * _L), jnp.float32),
    mesh=_mesh,
    scratch_types=[
        pltpu.VMEM((_CROWS, _COLS), jnp.float32),
        pltpu.VMEM((_CROWS, _COLS), jnp.float32),
        pltpu.VMEM((_CROWS, _COLS), jnp.float32),
        pltpu.VMEM((_CROWS, _COLS), jnp.float32),
        pltpu.VMEM((3 * _L,), jnp.float32),
        pltpu.SemaphoreType.DMA,
        pltpu.SemaphoreType.DMA,
        pltpu.SemaphoreType.DMA,
        pltpu.SemaphoreType.DMA,
    ],
)
def _sc_partials(x_hbm, t_hbm, out_hbm, xb0, xb1, tb0, tb1, accv,
                 sx0, sx1, st0, st1):
    wid = lax.axis_index("s") * _mesh.num_cores + lax.axis_index("c")
    base = _TC_ROWS + wid * _W_ROWS

    def start(c, xb, tb, sx, st):
        off = base + c * _CROWS
        pltpu.async_copy(x_hbm.at[pl.ds(off, _CROWS)], xb, sx)
        pltpu.async_copy(t_hbm.at[pl.ds(off, _CROWS)], tb, st)

    def wait(xb, tb, sx, st):
        pltpu.make_async_copy(x_hbm.at[pl.ds(0, _CROWS)], xb, sx).wait()
        pltpu.make_async_copy(t_hbm.at[pl.ds(0, _CROWS)], tb, st).wait()

    def consume(xb, tb, carry):
        for r in range(_CROWS):
            xbr = xb.at[r]
            tbr = tb.at[r]

            def body(j, car, xbr=xbr, tbr=tbr):
                a_all, a_z, a_n = car
                sqs, zs, ns = [], [], []
                for u in range(_UNROLL):
                    o = j * (_L * _UNROLL) + u * _L
                    xv = xbr[pl.ds(o, _L)]
                    tv = tbr[pl.ds(o, _L)]
                    d = xv - tv
                    sq = d * d
                    m = tv == 0.0
                    sqs.append(sq)
                    zs.append(jnp.where(m, sq, 0.0))
                    ns.append(jnp.where(m, 1.0, 0.0))
                return (a_all + _tree_sum(sqs),
                        a_z + _tree_sum(zs),
                        a_n + _tree_sum(ns))

            carry = lax.fori_loop(0, _COLS // (_L * _UNROLL), body, carry)
        return carry

    start(0, xb0, tb0, sx0, st0)
    zero = jnp.zeros((_L,), jnp.float32)

    def outer(g2, carry):
        start(2 * g2 + 1, xb1, tb1, sx1, st1)
        wait(xb0, tb0, sx0, st0)
        carry = consume(xb0, tb0, carry)

        @pl.when(2 * g2 + 2 < _SC_CHUNKS_PER_W)
        def _():
            start(2 * g2 + 2, xb0, tb0, sx0, st0)

        wait(xb1, tb1, sx1, st1)
        carry = consume(xb1, tb1, carry)
        return carry

    a_all, a_z, a_n = lax.fori_loop(0, _SC_CHUNKS_PER_W // 2, outer,
                                    (zero, zero, zero))
    accv[pl.ds(0, _L)] = a_all
    accv[pl.ds(_L, _L)] = a_z
    accv[pl.ds(2 * _L, _L)] = a_n
    pltpu.sync_copy(accv, out_hbm.at[wid])


def _tc_body(x_ref, t_ref, out_ref, acc_ref):
    i = pl.program_id(0)

    @pl.when(i == 0)
    def _init():
        acc_ref[0] = 0.0
        acc_ref[1] = 0.0
        acc_ref[2] = 0.0

    x = x_ref[...]
    t = t_ref[...]
    d = x - t
    sq = d * d
    zero = t == 0.0
    acc_ref[0] += jnp.sum(sq)
    acc_ref[1] += jnp.sum(jnp.where(zero, sq, 0.0))
    acc_ref[2] += jnp.sum(zero.astype(jnp.float32))

    @pl.when(i == pl.num_programs(0) - 1)
    def _fini():
        out_ref[0] = acc_ref[0]
        out_ref[1] = acc_ref[1]
        out_ref[2] = acc_ref[2]


def _fin_body(tc_ref, p_ref, out_ref):
    p = p_ref[...]
    s_all = tc_ref[0] + jnp.sum(p[:, 0:_L])
    s_z = tc_ref[1] + jnp.sum(p[:, _L:2 * _L])
    n_z = tc_ref[2] + jnp.sum(p[:, 2 * _L:3 * _L])
    n_total = float(_N)
    n_uz = n_total - n_z
    z_ratio = n_z / n_total
    loss_comp = s_all / n_total
    loss_z = s_z / jnp.maximum(n_z, 1.0)
    loss_uz = (s_all - s_z) / jnp.maximum(n_uz, 1.0)
    loss = loss_z * z_ratio * ZERO_WEIGHT + loss_uz * (1.0 - z_ratio)
    out_ref[0] = loss * (loss_comp / loss)


def kernel(input, target):
    x2 = input.reshape(_ROWS, _COLS)
    t2 = target.reshape(_ROWS, _COLS)

    sc_partials = _sc_partials(x2, t2)

    tc_partials = pl.pallas_call(
        _tc_body,
        grid=(_TC_GRID,),
        in_specs=[
            pl.BlockSpec((_TC_BLOCK_ROWS, _COLS), lambda i: (i, 0)),
            pl.BlockSpec((_TC_BLOCK_ROWS, _COLS), lambda i: (i, 0)),
        ],
        out_specs=pl.BlockSpec(memory_space=pltpu.SMEM),
        out_shape=jax.ShapeDtypeStruct((3,), jnp.float32),
        scratch_shapes=[pltpu.SMEM((3,), jnp.float32)],
    )(x2, t2)

    out = pl.pallas_call(
        _fin_body,
        in_specs=[
            pl.BlockSpec(memory_space=pltpu.SMEM),
            pl.BlockSpec((_NW, 3 * _L), lambda: (0, 0)),
        ],
        out_specs=pl.BlockSpec(memory_space=pltpu.SMEM),
        out_shape=jax.ShapeDtypeStruct((1,), jnp.float32),
    )(tc_partials, sc_partials)
    return out[0]


# speedup vs baseline: 1.1941x; 1.1941x over previous
import jax
import jax.numpy as jnp
from jax.experimental import pallas as pl
from jax.experimental.pallas import tpu as pltpu

ZERO_WEIGHT = 2.0

_ROWS = 2 * 8192
_COLS = 2048
_BLOCK_ROWS = 512
_GRID = _ROWS // _BLOCK_ROWS


def _body(x_ref, t_ref, out_ref, acc_ref):
    i = pl.program_id(0)

    @pl.when(i == 0)
    def _init():
        acc_ref[0] = 0.0
        acc_ref[1] = 0.0
        acc_ref[2] = 0.0

    x = x_ref[...]
    t = t_ref[...]
    d = x - t
    sq = d * d
    zero = t == 0.0
    acc_ref[0] += jnp.sum(sq)
    acc_ref[1] += jnp.sum(jnp.where(zero, sq, 0.0))
    acc_ref[2] += jnp.sum(zero.astype(jnp.float32))

    @pl.when(i == pl.num_programs(0) - 1)
    def _fini():
        n_total = float(_ROWS * _COLS)
        s_all = acc_ref[0]
        s_z = acc_ref[1]
        n_z = acc_ref[2]
        n_uz = n_total - n_z
        z_ratio = n_z / n_total
        loss_comp = s_all / n_total
        loss_z = s_z / jnp.maximum(n_z, 1.0)
        loss_uz = (s_all - s_z) / jnp.maximum(n_uz, 1.0)
        loss = loss_z * z_ratio * ZERO_WEIGHT + loss_uz * (1.0 - z_ratio)
        out_ref[0] = loss * (loss_comp / loss)


def kernel(input, target):
    x = input.reshape(_ROWS, _COLS)
    t = target.reshape(_ROWS, _COLS)
    out = pl.pallas_call(
        _body,
        grid=(_GRID,),
        in_specs=[
            pl.BlockSpec((_BLOCK_ROWS, _COLS), lambda i: (i, 0)),
            pl.BlockSpec((_BLOCK_ROWS, _COLS), lambda i: (i, 0)),
        ],
        out_specs=pl.BlockSpec(memory_space=pltpu.SMEM),
        out_shape=jax.ShapeDtypeStruct((1,), jnp.float32),
        scratch_shapes=[pltpu.SMEM((3,), jnp.float32)],
    )(x, t)
    return out[0]


# speedup vs baseline: 1.2483x; 1.0454x over previous
import jax
import jax.numpy as jnp
from jax.experimental import pallas as pl
from jax.experimental.pallas import tpu as pltpu

ZERO_WEIGHT = 2.0

_ROWS = 2 * 8192
_COLS = 2048
_BLOCK_ROWS = 1024
_GRID = _ROWS // _BLOCK_ROWS


def _body(x_ref, t_ref, out_ref, acc_ref):
    i = pl.program_id(0)

    @pl.when(i == 0)
    def _init():
        acc_ref[0] = 0.0
        acc_ref[1] = 0.0
        acc_ref[2] = 0.0

    x = x_ref[...]
    t = t_ref[...]
    d = x - t
    sq = d * d
    zero = t == 0.0
    acc_ref[0] += jnp.sum(sq)
    acc_ref[1] += jnp.sum(jnp.where(zero, sq, 0.0))
    acc_ref[2] += jnp.sum(zero.astype(jnp.float32))

    @pl.when(i == pl.num_programs(0) - 1)
    def _fini():
        n_total = float(_ROWS * _COLS)
        s_all = acc_ref[0]
        s_z = acc_ref[1]
        n_z = acc_ref[2]
        n_uz = n_total - n_z
        z_ratio = n_z / n_total
        loss_comp = s_all / n_total
        loss_z = s_z / jnp.maximum(n_z, 1.0)
        loss_uz = (s_all - s_z) / jnp.maximum(n_uz, 1.0)
        loss = loss_z * z_ratio * ZERO_WEIGHT + loss_uz * (1.0 - z_ratio)
        out_ref[0] = loss * (loss_comp / loss)


def kernel(input, target):
    x = input.reshape(_ROWS, _COLS)
    t = target.reshape(_ROWS, _COLS)
    out = pl.pallas_call(
        _body,
        grid=(_GRID,),
        in_specs=[
            pl.BlockSpec((_BLOCK_ROWS, _COLS), lambda i: (i, 0)),
            pl.BlockSpec((_BLOCK_ROWS, _COLS), lambda i: (i, 0)),
        ],
        out_specs=pl.BlockSpec(memory_space=pltpu.SMEM),
        out_shape=jax.ShapeDtypeStruct((1,), jnp.float32),
        scratch_shapes=[pltpu.SMEM((3,), jnp.float32)],
    )(x, t)
    return out[0]
